# baseline (speedup 1.0000x reference)
import jax
import jax.numpy as jnp
from jax.experimental import pallas as pl


def _gat_body(x_ref, adj_ref, w_ref, a1t_ref, a2t_ref, o_ref):
    x = x_ref[0]
    h = jnp.dot(x, w_ref[...], preferred_element_type=jnp.float32)
    mask = (adj_ref[...] > 0).astype(jnp.float32)
    h2 = jax.lax.dot_general(
        mask, h, (((0,), (0,)), ((), ())), preferred_element_type=jnp.float32
    )
    score = (
        jnp.sum(h * a1t_ref[...], axis=1) + jnp.sum(h2 * a2t_ref[...], axis=1)
    )
    hsum = jnp.sum(h, axis=0)
    o_ref[0] = jnp.maximum(score[:, None] * hsum[None, :], 0.0)


def kernel(inp, adj, W, a):
    b, t, n, fin = inp.shape
    fout = W.shape[1]
    bt = b * t
    x = inp.reshape(bt, n, fin)
    a1t = a[:fout, :].T
    a2t = a[fout:, :].T

    out = pl.pallas_call(
        _gat_body,
        grid=(bt,),
        in_specs=[
            pl.BlockSpec((1, n, fin), lambda i: (i, 0, 0)),
            pl.BlockSpec((n, n), lambda i: (0, 0)),
            pl.BlockSpec((fin, fout), lambda i: (0, 0)),
            pl.BlockSpec((n, fout), lambda i: (0, 0)),
            pl.BlockSpec((n, fout), lambda i: (0, 0)),
        ],
        out_specs=pl.BlockSpec((1, n, fout), lambda i: (i, 0, 0)),
        out_shape=jax.ShapeDtypeStruct((bt, n, fout), jnp.float32),
    )(x, adj, W, a1t, a2t)
    return out.reshape(b, t, n, fout)
